# baseline (speedup 1.0000x reference)
import functools

import numpy as np

import jax
import jax.numpy as jnp
from jax.experimental import pallas as pl
from jax.experimental.pallas import tpu as pltpu

_NC = 2
_SUBS = (256, 256, 256, 96)


def _ru(x, m):
    return -(-x // m) * m


def _geoms(b2, d0, h0, w0, chans):
    gs = []
    d, h, w = d0, h0, w0
    for i, (cin, cout) in enumerate(chans):
        dp, hp, wp = d + 2, h + 2, w + 2
        sub = _SUBS[i]
        mp = b2 * dp * hp
        n_tiles = -(-mp // sub)
        mcomp = n_tiles * sub
        rows = _ru(mcomp + 2 * hp + 2, 8)
        gs.append(dict(
            d=d, h=h, w=w, cin=cin, cout=cout, dp=dp, hp=hp, wp=wp,
            sub=sub, n_tiles=n_tiles, mcomp=mcomp, rows=rows,
            K=wp * cin, N=w * cout,
            offs=tuple(kd * hp + kh for kd in range(3) for kh in range(3)),
        ))
        d, h, w = d // 2, h // 2, w // 2
    return gs


def _net_kernel(xin_ref, b0, s0, a0, b1, s1, a1, b2_, s2, a2, b3, s3, a3,
                p_ref, tab_ref, t1w, t1s, t1b, t2w, t2b,
                c1wi, c1wt, c1b, c2w, c2b,
                o_ref, x0, x1, x2, x3, y0, y1, y2, *, G, B2):
    f32, bf16 = jnp.float32, jnp.bfloat16
    bands = (b0, b1, b2_, b3)
    scales = (s0, s1, s2, s3)
    biases = (a0, a1, a2, a3)
    xs = (x0, x1, x2, x3)
    ys = (y0, y1, y2, None)

    x0[...] = jnp.zeros_like(x0)
    x1[...] = jnp.zeros_like(x1)
    x2[...] = jnp.zeros_like(x2)
    x3[...] = jnp.zeros_like(x3)

    g0 = G[0]
    cin0 = g0["cin"]
    for b in range(B2):
        for d in range(g0["d"]):
            row0 = (b * g0["dp"] + d + 1) * g0["hp"] + 1
            x0[row0:row0 + g0["h"], cin0:cin0 + g0["w"] * cin0] = (
                xin_ref[b, d].astype(bf16))

    def ld(ref, lo, n):
        if ref.ndim == 3:
            return ref[0, pl.ds(lo, n), :]
        return ref[pl.ds(lo, n), :]

    img = None
    for i in range(4):
        g = G[i]
        sub, offs, N = g["sub"], g["offs"], g["N"]
        scale = scales[i][...]
        bias = biases[i][...]
        band = bands[i]
        xr = xs[i]

        ext = sub + 2 * g["hp"] + 2
        if i < 3:
            yr = ys[i]

            def conv_body(ti, _, xr=xr, band=band, scale=scale, bias=bias,
                          yr=yr, sub=sub, offs=offs, N=N, ext=ext):
                r0 = ti * sub
                slab = ld(xr, r0, ext)
                acc = jnp.zeros((sub, N), f32)
                for t, off in enumerate(offs):
                    acc = acc + jnp.dot(slab[off:off + sub, :], band[t],
                                        preferred_element_type=f32)
                yr[pl.ds(r0, sub), :] = jnp.maximum(
                    acc * scale + bias, 0.0).astype(bf16)
                return 0

            jax.lax.fori_loop(0, g["n_tiles"], conv_body, 0)

            gn = G[i + 1]
            H, D, Hp, Dp = g["h"], g["d"], g["hp"], g["dp"]
            half = N // 2
            cout = g["cout"]
            hp2, dp2 = gn["hp"], gn["dp"]
            yr = ys[i]
            xn = xs[i + 1]
            for b in range(B2):
                for d2 in range(D // 2):
                    rb = (b * Dp + 2 * d2) * Hp
                    yd = jnp.maximum(yr[rb:rb + H, :], yr[rb + Hp:rb + Hp + H, :])
                    yw = jnp.maximum(yd[:, :half], yd[:, half:])
                    v = yw.reshape(H // 2, 2 * half)
                    zh = jnp.maximum(v[:, :half], v[:, half:])
                    orow = (b * dp2 + d2 + 1) * hp2 + 1
                    xn[orow:orow + H // 2, cout:cout + half] = zh
        else:
            img_acc = jnp.zeros((B2, N), f32)
            for ti in range(g["n_tiles"]):
                r0 = ti * sub
                acc = jnp.zeros((sub, N), f32)
                for t, off in enumerate(offs):
                    acc = acc + jnp.dot(xr[r0 + off:r0 + off + sub, :], band[t],
                                        preferred_element_type=f32)
                yt = jnp.maximum(acc * scale + bias, 0.0).astype(bf16)
                img_acc = img_acc + jnp.dot(p_ref[:, r0:r0 + sub], yt,
                                            preferred_element_type=f32)
            img = img_acc[:, :N // 2] + img_acc[:, N // 2:]

    t = jnp.dot(tab_ref[0], t1w[...], preferred_element_type=f32)
    t = jnp.maximum(t * t1s[...] + t1b[...], 0.0)
    t = jnp.dot(t.astype(bf16), t2w[...], preferred_element_type=f32) + t2b[...]
    t = jnp.maximum(t, 0.0)

    h = (jnp.dot(img.astype(bf16), c1wi[...], preferred_element_type=f32)
         + jnp.dot(t.astype(bf16), c1wt[...], preferred_element_type=f32)
         + c1b[...])
    h = jnp.maximum(h, 0.0)
    out = jnp.dot(h.astype(bf16), c2w[...], preferred_element_type=f32) + c2b[...]
    o_ref[0] = out.astype(o_ref.dtype)


def _zmap(nd):
    return lambda i, _n=nd: (0,) * _n


def kernel(conv0_band, conv0_scale, conv0_bias, conv1_band, conv1_scale,
           conv1_bias, conv2_band, conv2_scale, conv2_bias, conv3_band,
           conv3_scale, conv3_bias, head_t1w, head_t1s, head_t1b, head_t2w,
           head_t2b, head_c1wi, head_c1wt, head_c1b, head_c2w, head_c2b,
           x_img, x_tab):
    B, D, H, W, Cin = x_img.shape
    B2 = B // _NC
    w_sizes = [W, W // 2, W // 4, W // 8]
    bandsin = [conv0_band, conv1_band, conv2_band, conv3_band]
    chans, cin = [], Cin
    for i in range(4):
        cout = bandsin[i].shape[2] // w_sizes[i]
        chans.append((cin, cout))
        cin = cout
    G = _geoms(B2, D, H, W, chans)
    g0, g3 = G[0], G[3]

    rows = x_img.astype(jnp.bfloat16).reshape(B, D, H, W * Cin)

    m3 = g3["mcomp"]
    pnp = np.zeros((B2, m3), np.float32)
    inv = 1.0 / float(g3["d"] * g3["h"] * g3["w"])
    for b in range(B2):
        for d in range(g3["d"]):
            for h in range(g3["h"]):
                pnp[b, (b * g3["dp"] + d) * g3["hp"] + h] = inv
    pmat = jnp.asarray(pnp, jnp.bfloat16)

    tab = x_tab.astype(jnp.bfloat16).reshape(_NC, B2, x_tab.shape[1])

    args = [rows, conv0_band, conv0_scale, conv0_bias,
            conv1_band, conv1_scale, conv1_bias,
            conv2_band, conv2_scale, conv2_bias,
            conv3_band, conv3_scale, conv3_bias,
            pmat, tab, head_t1w, head_t1s, head_t1b, head_t2w, head_t2b,
            head_c1wi, head_c1wt, head_c1b, head_c2w, head_c2b]

    in_specs = [pl.BlockSpec((B2, D, H, W * Cin), lambda i: (i, 0, 0, 0))]
    for a in args[1:]:
        if a is tab:
            in_specs.append(pl.BlockSpec((1,) + tab.shape[1:],
                                         lambda i: (i, 0, 0)))
        else:
            in_specs.append(pl.BlockSpec(a.shape, _zmap(a.ndim)))

    out = pl.pallas_call(
        functools.partial(_net_kernel, G=G, B2=B2),
        out_shape=jax.ShapeDtypeStruct((_NC, B2, 1), jnp.float32),
        grid=(_NC,),
        in_specs=in_specs,
        out_specs=pl.BlockSpec((1, B2, 1), lambda i: (i, 0, 0)),
        scratch_shapes=[
            pltpu.VMEM((G[0]["rows"], G[0]["K"]), jnp.bfloat16),
            pltpu.VMEM((G[1]["rows"], G[1]["K"]), jnp.bfloat16),
            pltpu.VMEM((G[2]["rows"], G[2]["K"]), jnp.bfloat16),
            pltpu.VMEM((G[3]["rows"], G[3]["K"]), jnp.bfloat16),
            pltpu.VMEM((G[0]["mcomp"], G[0]["N"]), jnp.bfloat16),
            pltpu.VMEM((G[1]["mcomp"], G[1]["N"]), jnp.bfloat16),
            pltpu.VMEM((G[2]["mcomp"], G[2]["N"]), jnp.bfloat16),
        ],
        compiler_params=pltpu.CompilerParams(
            dimension_semantics=("arbitrary",),
            vmem_limit_bytes=100 * 1024 * 1024,
        ),
    )(*args)
    return out.reshape(B, 1)


# speedup vs baseline: 1.0885x; 1.0885x over previous
import functools

import numpy as np

import jax
import jax.numpy as jnp
from jax.experimental import pallas as pl
from jax.experimental.pallas import tpu as pltpu

_NC = 2
_SUBS = (256, 256, 256, 96)


def _ru(x, m):
    return -(-x // m) * m


def _geoms(b2, d0, h0, w0, chans):
    gs = []
    d, h, w = d0, h0, w0
    for i, (cin, cout) in enumerate(chans):
        dp, hp, wp = d + 2, h + 2, w + 2
        sub = _SUBS[i]
        unroll = 2 if i < 3 else 1
        mp = b2 * dp * hp
        n_bodies = -(-mp // (sub * unroll))
        n_tiles = n_bodies * unroll
        mcomp = n_tiles * sub
        rows = _ru(mcomp + 2 * hp + 2, 8)
        gs.append(dict(
            d=d, h=h, w=w, cin=cin, cout=cout, dp=dp, hp=hp, wp=wp,
            sub=sub, unroll=unroll, n_bodies=n_bodies,
            n_tiles=n_tiles, mcomp=mcomp, rows=rows,
            K=wp * cin, N=w * cout,
            offs=tuple(kd * hp + kh for kd in range(3) for kh in range(3)),
        ))
        d, h, w = d // 2, h // 2, w // 2
    return gs


def _net_kernel(xin_ref, b0, s0, a0, b1, s1, a1, b2_, s2, a2, b3, s3, a3,
                p_ref, tab_ref, t1w, t1s, t1b, t2w, t2b,
                c1wi, c1wt, c1b, c2w, c2b,
                o_ref, x0, x1, x2, x3, y0, y1, y2, *, G, B2):
    f32, bf16 = jnp.float32, jnp.bfloat16
    bands = (b0, b1, b2_, b3)
    scales = (s0, s1, s2, s3)
    biases = (a0, a1, a2, a3)
    xs = (x0, x1, x2, x3)
    ys = (y0, y1, y2, None)

    x0[...] = jnp.zeros_like(x0)
    x1[...] = jnp.zeros_like(x1)
    x2[...] = jnp.zeros_like(x2)
    x3[...] = jnp.zeros_like(x3)

    g0 = G[0]
    cin0 = g0["cin"]
    for b in range(B2):
        for d in range(g0["d"]):
            row0 = (b * g0["dp"] + d + 1) * g0["hp"] + 1
            x0[row0:row0 + g0["h"], cin0:cin0 + g0["w"] * cin0] = (
                xin_ref[b, d].astype(bf16))

    def ld(ref, lo, n):
        if ref.ndim == 3:
            return ref[0, pl.ds(lo, n), :]
        return ref[pl.ds(lo, n), :]

    img = None
    for i in range(4):
        g = G[i]
        sub, offs, N = g["sub"], g["offs"], g["N"]
        scale = scales[i][...]
        bias = biases[i][...]
        band = bands[i]
        xr = xs[i]

        ext = sub + 2 * g["hp"] + 2
        if i < 3:
            yr = ys[i]

            def conv_body(bi, _, xr=xr, band=band, scale=scale, bias=bias,
                          yr=yr, sub=sub, offs=offs, N=N, ext=ext,
                          U=g["unroll"]):
                for u in range(U):
                    r0 = (bi * U + u) * sub
                    slab = ld(xr, r0, ext)
                    acc = jnp.zeros((sub, N), f32)
                    for t, off in enumerate(offs):
                        acc = acc + jnp.dot(slab[off:off + sub, :], band[t],
                                            preferred_element_type=f32)
                    yr[pl.ds(r0, sub), :] = jnp.maximum(
                        acc * scale + bias, 0.0).astype(bf16)
                return 0

            jax.lax.fori_loop(0, g["n_bodies"], conv_body, 0)

            gn = G[i + 1]
            H, D, Hp, Dp = g["h"], g["d"], g["hp"], g["dp"]
            half = N // 2
            cout = g["cout"]
            hp2, dp2 = gn["hp"], gn["dp"]
            yr = ys[i]
            xn = xs[i + 1]
            for b in range(B2):
                for d2 in range(D // 2):
                    rb = (b * Dp + 2 * d2) * Hp
                    yd = jnp.maximum(yr[rb:rb + H, :], yr[rb + Hp:rb + Hp + H, :])
                    yw = jnp.maximum(yd[:, :half], yd[:, half:])
                    v = yw.reshape(H // 2, 2 * half)
                    zh = jnp.maximum(v[:, :half], v[:, half:])
                    orow = (b * dp2 + d2 + 1) * hp2 + 1
                    xn[orow:orow + H // 2, cout:cout + half] = zh
        else:
            img_acc = jnp.zeros((B2, N), f32)
            for ti in range(g["n_tiles"]):
                r0 = ti * sub
                acc = jnp.zeros((sub, N), f32)
                for t, off in enumerate(offs):
                    acc = acc + jnp.dot(xr[r0 + off:r0 + off + sub, :], band[t],
                                        preferred_element_type=f32)
                yt = jnp.maximum(acc * scale + bias, 0.0).astype(bf16)
                img_acc = img_acc + jnp.dot(p_ref[:, r0:r0 + sub], yt,
                                            preferred_element_type=f32)
            img = img_acc[:, :N // 2] + img_acc[:, N // 2:]

    t = jnp.dot(tab_ref[0], t1w[...], preferred_element_type=f32)
    t = jnp.maximum(t * t1s[...] + t1b[...], 0.0)
    t = jnp.dot(t.astype(bf16), t2w[...], preferred_element_type=f32) + t2b[...]
    t = jnp.maximum(t, 0.0)

    h = (jnp.dot(img.astype(bf16), c1wi[...], preferred_element_type=f32)
         + jnp.dot(t.astype(bf16), c1wt[...], preferred_element_type=f32)
         + c1b[...])
    h = jnp.maximum(h, 0.0)
    out = jnp.dot(h.astype(bf16), c2w[...], preferred_element_type=f32) + c2b[...]
    o_ref[0] = out.astype(o_ref.dtype)


def _zmap(nd):
    return lambda i, _n=nd: (0,) * _n


def kernel(conv0_band, conv0_scale, conv0_bias, conv1_band, conv1_scale,
           conv1_bias, conv2_band, conv2_scale, conv2_bias, conv3_band,
           conv3_scale, conv3_bias, head_t1w, head_t1s, head_t1b, head_t2w,
           head_t2b, head_c1wi, head_c1wt, head_c1b, head_c2w, head_c2b,
           x_img, x_tab):
    B, D, H, W, Cin = x_img.shape
    B2 = B // _NC
    w_sizes = [W, W // 2, W // 4, W // 8]
    bandsin = [conv0_band, conv1_band, conv2_band, conv3_band]
    chans, cin = [], Cin
    for i in range(4):
        cout = bandsin[i].shape[2] // w_sizes[i]
        chans.append((cin, cout))
        cin = cout
    G = _geoms(B2, D, H, W, chans)
    g0, g3 = G[0], G[3]

    rows = x_img.reshape(B, D, H, W * Cin)

    m3 = g3["mcomp"]
    pnp = np.zeros((B2, m3), np.float32)
    inv = 1.0 / float(g3["d"] * g3["h"] * g3["w"])
    for b in range(B2):
        for d in range(g3["d"]):
            for h in range(g3["h"]):
                pnp[b, (b * g3["dp"] + d) * g3["hp"] + h] = inv
    pmat = jnp.asarray(pnp, jnp.bfloat16)

    tab = x_tab.astype(jnp.bfloat16).reshape(_NC, B2, x_tab.shape[1])

    args = [rows, conv0_band, conv0_scale, conv0_bias,
            conv1_band, conv1_scale, conv1_bias,
            conv2_band, conv2_scale, conv2_bias,
            conv3_band, conv3_scale, conv3_bias,
            pmat, tab, head_t1w, head_t1s, head_t1b, head_t2w, head_t2b,
            head_c1wi, head_c1wt, head_c1b, head_c2w, head_c2b]

    in_specs = [pl.BlockSpec((B2, D, H, W * Cin), lambda i: (i, 0, 0, 0))]
    for a in args[1:]:
        if a is tab:
            in_specs.append(pl.BlockSpec((1,) + tab.shape[1:],
                                         lambda i: (i, 0, 0)))
        else:
            in_specs.append(pl.BlockSpec(a.shape, _zmap(a.ndim)))

    out = pl.pallas_call(
        functools.partial(_net_kernel, G=G, B2=B2),
        out_shape=jax.ShapeDtypeStruct((_NC, B2, 1), jnp.float32),
        grid=(_NC,),
        in_specs=in_specs,
        out_specs=pl.BlockSpec((1, B2, 1), lambda i: (i, 0, 0)),
        scratch_shapes=[
            pltpu.VMEM((G[0]["rows"], G[0]["K"]), jnp.bfloat16),
            pltpu.VMEM((G[1]["rows"], G[1]["K"]), jnp.bfloat16),
            pltpu.VMEM((G[2]["rows"], G[2]["K"]), jnp.bfloat16),
            pltpu.VMEM((G[3]["rows"], G[3]["K"]), jnp.bfloat16),
            pltpu.VMEM((G[0]["mcomp"], G[0]["N"]), jnp.bfloat16),
            pltpu.VMEM((G[1]["mcomp"], G[1]["N"]), jnp.bfloat16),
            pltpu.VMEM((G[2]["mcomp"], G[2]["N"]), jnp.bfloat16),
        ],
        compiler_params=pltpu.CompilerParams(
            dimension_semantics=("arbitrary",),
            vmem_limit_bytes=100 * 1024 * 1024,
        ),
    )(*args)
    return out.reshape(B, 1)
